# baseline (speedup 1.0000x reference)
import functools

import jax
import jax.numpy as jnp
from jax import lax
from jax.experimental import pallas as pl
from jax.experimental.pallas import tpu as pltpu
from jax.experimental.pallas import tpu_sc as plsc

DIM = 128
N_NODES = 512
N_EDGES = 8192
N_ROWS = N_NODES * N_NODES
NC, NS = 2, 16
NW = NC * NS
R = N_ROWS // NW
PB = 256
NBLK = R // PB
LANES = 16
DMASK = N_ROWS - 1


def _prep_body(nn_ref, ei_ref, feat_ref, t0_ref, t1_ref, t2_ref,
               ctable_ref, pk_ref):
    r = lax.broadcasted_iota(jnp.int32, (64, 1), 0)
    j0 = lax.broadcasted_iota(jnp.int32, (64, 5), 1)
    j1 = lax.broadcasted_iota(jnp.int32, (64, 6), 1)
    j2 = lax.broadcasted_iota(jnp.int32, (64, 2), 1)
    oh0 = (j0 == r // 12).astype(jnp.float32)
    oh1 = (j1 == (r // 2) % 6).astype(jnp.float32)
    oh2 = (j2 == r % 2).astype(jnp.float32)
    ctable_ref[...] = (
        jnp.dot(oh0, t0_ref[...], preferred_element_type=jnp.float32,
                precision=lax.Precision.HIGHEST)
        + jnp.dot(oh1, t1_ref[...], preferred_element_type=jnp.float32,
                  precision=lax.Precision.HIGHEST)
        + jnp.dot(oh2, t2_ref[...], preferred_element_type=jnp.float32,
                  precision=lax.Precision.HIGHEST)
    )
    off = nn_ref[0] - N_NODES
    x = ei_ref[0, :] + off
    y = ei_ref[1, :] + off
    f = feat_ref[...]
    cidx = f[:, 0] * 12 + f[:, 1] * 2 + f[:, 2]
    pk_ref[...] = (x * N_NODES + y) | lax.shift_left(cidx, 18)


_prep = pl.pallas_call(
    _prep_body,
    in_specs=[
        pl.BlockSpec(memory_space=pltpu.SMEM),
        pl.BlockSpec(),
        pl.BlockSpec(),
        pl.BlockSpec(),
        pl.BlockSpec(),
        pl.BlockSpec(),
    ],
    out_shape=(
        jax.ShapeDtypeStruct((64, DIM), jnp.float32),
        jax.ShapeDtypeStruct((N_EDGES,), jnp.int32),
    ),
)


@functools.partial(
    pl.kernel,
    out_type=jax.ShapeDtypeStruct((N_ROWS, DIM), jnp.float32),
    mesh=plsc.VectorSubcoreMesh(
        core_axis_name="c", subcore_axis_name="s", num_cores=NC, num_subcores=NS
    ),
    compiler_params=pltpu.CompilerParams(needs_layout_passes=False),
    scratch_types=[
        pltpu.VMEM((PB, DIM), jnp.float32),
        pltpu.VMEM((PB, DIM), jnp.float32),
        pltpu.VMEM((64, DIM), jnp.float32),
        pltpu.VMEM((N_EDGES,), jnp.int32),
        pltpu.VMEM((N_EDGES,), jnp.int32),
        pltpu.VMEM((N_EDGES,), jnp.int32),
        pltpu.VMEM((N_EDGES,), jnp.int32),
        pltpu.SemaphoreType.DMA,
        pltpu.SemaphoreType.DMA,
    ],
)
def _sc_fill_scatter(pk_hbm, ctable_hbm, pad_hbm, out_hbm,
                     buf_a, buf_b, ctable_v, pk_v, myent, blk_a, blk_b,
                     sem_a, sem_b):
    wid = lax.axis_index("c") * NS + lax.axis_index("s")
    row0 = wid * R
    lo = row0
    hi = row0 + R

    iota = lax.iota(jnp.int32, LANES)
    zeros = iota * 0

    pltpu.sync_copy(pad_hbm, buf_a.at[0])
    pvs = [buf_a[0, pl.ds(d * LANES, LANES)] for d in range(DIM // LANES)]

    def _fill_rows(buf, start):
        def _fill_row(rr, carry):
            for d in range(DIM // LANES):
                buf[rr, pl.ds(d * LANES, LANES)] = pvs[d]
            return carry
        lax.fori_loop(start, PB, _fill_row, 0)

    _fill_rows(buf_a, 1)
    _fill_rows(buf_b, 0)

    pltpu.sync_copy(ctable_hbm, ctable_v)
    pltpu.sync_copy(pk_hbm, pk_v)

    def _compact(i, off):
        pk = pk_v[pl.ds(i * LANES, LANES)]
        v = pk & DMASK
        m = (v >= lo) & (v < hi)
        mi = m.astype(jnp.int32)
        pos = jnp.maximum(off + plsc.cumsum(mi) - 1, 0)
        plsc.store_scatter(myent, [pos], pk, mask=m)
        return off + jnp.sum(mi)

    n = lax.fori_loop(0, N_EDGES // LANES, _compact, jnp.int32(0))
    nit = lax.shift_right_logical(n + (LANES - 1), 4)

    def _blk_compact(blk, dstlist):
        def body(i, carry):
            off, ev = carry
            pk = myent[pl.ds(i * LANES, LANES)]
            d = pk & DMASK
            m = ((lax.shift_right_logical(d - lo, 8) == blk) & (ev < n))
            mi = m.astype(jnp.int32)
            pos = jnp.maximum(off + plsc.cumsum(mi) - 1, 0)
            plsc.store_scatter(dstlist, [pos], pk, mask=m)
            return off + jnp.sum(mi), ev + LANES

        nb, _ = lax.fori_loop(0, nit, body, (jnp.int32(0), iota))
        return nb

    def _patch(buf, lst, nb):
        def body(e, esplat):
            pks = plsc.load_gather(lst, [esplat])[0]
            rr = pks & (PB - 1)
            cc = lax.shift_right_logical(pks, 18)
            for d in range(DIM // LANES):
                buf[rr, pl.ds(d * LANES, LANES)] = (
                    ctable_v[cc, pl.ds(d * LANES, LANES)])
            return esplat + 1
        lax.fori_loop(0, nb, body, zeros)

    def _unpatch(buf, lst, nb):
        def body(e, esplat):
            pks = plsc.load_gather(lst, [esplat])[0]
            rr = pks & (PB - 1)
            for d in range(DIM // LANES):
                buf[rr, pl.ds(d * LANES, LANES)] = pvs[d]
            return esplat + 1
        lax.fori_loop(0, nb, body, zeros)

    def _fire(buf, blk_scalar, sem):
        pltpu.async_copy(
            buf, out_hbm.at[pl.ds(row0 + blk_scalar * PB, PB)], sem)

    def _drain(buf, sem):
        pltpu.make_async_copy(buf, out_hbm.at[pl.ds(row0, PB)], sem).wait()

    na0 = _blk_compact(zeros, blk_a)
    _patch(buf_a, blk_a, na0)
    _fire(buf_a, jnp.int32(0), sem_a)
    nb0 = _blk_compact(zeros + 1, blk_b)
    _patch(buf_b, blk_b, nb0)
    _fire(buf_b, jnp.int32(1), sem_b)

    def _step(t, carry):
        na_p, nb_p, bva, bvb = carry
        bva = bva + 2
        bvb = bvb + 2
        _drain(buf_a, sem_a)
        _unpatch(buf_a, blk_a, na_p)
        na = _blk_compact(bva, blk_a)
        _patch(buf_a, blk_a, na)
        _fire(buf_a, bva[0], sem_a)
        _drain(buf_b, sem_b)
        _unpatch(buf_b, blk_b, nb_p)
        nbn = _blk_compact(bvb, blk_b)
        _patch(buf_b, blk_b, nbn)
        _fire(buf_b, bvb[0], sem_b)
        return na, nbn, bva, bvb

    lax.fori_loop(1, NBLK // 2, _step, (na0, nb0, zeros, zeros + 1))

    _drain(buf_a, sem_a)
    _drain(buf_b, sem_b)


def kernel(edge_index, edge_feat, num_nodes, padding_emb, table0, table1, table2):
    nn = jnp.asarray(num_nodes, jnp.int32).reshape(1)
    ctable, pk = _prep(nn, edge_index.astype(jnp.int32),
                       edge_feat.astype(jnp.int32), table0, table1, table2)
    out = _sc_fill_scatter(pk, ctable, padding_emb)
    return out.reshape(N_NODES, N_NODES, DIM)


# speedup vs baseline: 1.0023x; 1.0023x over previous
import functools

import jax
import jax.numpy as jnp
from jax import lax
from jax.experimental import pallas as pl
from jax.experimental.pallas import tpu as pltpu
from jax.experimental.pallas import tpu_sc as plsc

DIM = 128
N_NODES = 512
N_EDGES = 8192
N_ROWS = N_NODES * N_NODES
NC, NS = 2, 16
NW = NC * NS
R = N_ROWS // NW
PB = 256
NBLK = R // PB
LANES = 16
DMASK = N_ROWS - 1


def _prep_body(nn_ref, ei_ref, feat_ref, t0_ref, t1_ref, t2_ref,
               ctable_ref, pk_ref):
    r = lax.broadcasted_iota(jnp.int32, (64, 1), 0)
    j0 = lax.broadcasted_iota(jnp.int32, (64, 5), 1)
    j1 = lax.broadcasted_iota(jnp.int32, (64, 6), 1)
    j2 = lax.broadcasted_iota(jnp.int32, (64, 2), 1)
    oh0 = (j0 == r // 12).astype(jnp.float32)
    oh1 = (j1 == (r // 2) % 6).astype(jnp.float32)
    oh2 = (j2 == r % 2).astype(jnp.float32)
    ctable_ref[...] = (
        jnp.dot(oh0, t0_ref[...], preferred_element_type=jnp.float32,
                precision=lax.Precision.HIGHEST)
        + jnp.dot(oh1, t1_ref[...], preferred_element_type=jnp.float32,
                  precision=lax.Precision.HIGHEST)
        + jnp.dot(oh2, t2_ref[...], preferred_element_type=jnp.float32,
                  precision=lax.Precision.HIGHEST)
    )
    off = nn_ref[0] - N_NODES
    x = ei_ref[0, :] + off
    y = ei_ref[1, :] + off
    f = feat_ref[...]
    cidx = f[:, 0] * 12 + f[:, 1] * 2 + f[:, 2]
    pk_ref[...] = (x * N_NODES + y) | lax.shift_left(cidx, 18)


_prep = pl.pallas_call(
    _prep_body,
    in_specs=[
        pl.BlockSpec(memory_space=pltpu.SMEM),
        pl.BlockSpec(),
        pl.BlockSpec(),
        pl.BlockSpec(),
        pl.BlockSpec(),
        pl.BlockSpec(),
    ],
    out_shape=(
        jax.ShapeDtypeStruct((64, DIM), jnp.float32),
        jax.ShapeDtypeStruct((N_EDGES,), jnp.int32),
    ),
)


@functools.partial(
    pl.kernel,
    out_type=jax.ShapeDtypeStruct((N_ROWS, DIM), jnp.float32),
    mesh=plsc.VectorSubcoreMesh(
        core_axis_name="c", subcore_axis_name="s", num_cores=NC, num_subcores=NS
    ),
    compiler_params=pltpu.CompilerParams(needs_layout_passes=False),
    scratch_types=[
        pltpu.VMEM((PB, DIM), jnp.float32),
        pltpu.VMEM((PB, DIM), jnp.float32),
        pltpu.VMEM((64, DIM), jnp.float32),
        pltpu.VMEM((N_EDGES,), jnp.int32),
        pltpu.VMEM((N_EDGES,), jnp.int32),
        pltpu.VMEM((N_EDGES,), jnp.int32),
        pltpu.VMEM((N_EDGES,), jnp.int32),
        pltpu.SemaphoreType.DMA,
        pltpu.SemaphoreType.DMA,
    ],
)
def _sc_fill_scatter(pk_hbm, ctable_hbm, pad_hbm, out_hbm,
                     buf_a, buf_b, ctable_v, pk_v, myent, blk_a, blk_b,
                     sem_a, sem_b):
    wid = lax.axis_index("c") * NS + lax.axis_index("s")
    row0 = wid * R
    lo = row0
    hi = row0 + R

    iota = lax.iota(jnp.int32, LANES)
    zeros = iota * 0

    pltpu.sync_copy(pad_hbm, buf_a.at[0])
    pvs = [buf_a[0, pl.ds(d * LANES, LANES)] for d in range(DIM // LANES)]

    def _fill_rows(buf, start):
        def _fill_row(rr, carry):
            for d in range(DIM // LANES):
                buf[rr, pl.ds(d * LANES, LANES)] = pvs[d]
            return carry
        lax.fori_loop(start, PB, _fill_row, 0)

    _fill_rows(buf_a, 1)
    _fill_rows(buf_b, 0)

    pltpu.sync_copy(ctable_hbm, ctable_v)
    pltpu.sync_copy(pk_hbm, pk_v)

    def _compact(i, off):
        pk = pk_v[pl.ds(i * LANES, LANES)]
        v = pk & DMASK
        m = (v >= lo) & (v < hi)
        mi = m.astype(jnp.int32)
        cs = plsc.cumsum(mi)
        pos = jnp.maximum(off + cs - 1, 0)
        plsc.store_scatter(myent, [pos], pk, mask=m)
        return off + cs[LANES - 1]

    n = lax.fori_loop(0, N_EDGES // LANES, _compact, jnp.int32(0))
    nit = lax.shift_right_logical(n + (LANES - 1), 4)

    def _blk_compact(blk, dstlist):
        def body(i, carry):
            off, ev = carry
            pk = myent[pl.ds(i * LANES, LANES)]
            d = pk & DMASK
            m = ((lax.shift_right_logical(d - lo, 8) == blk) & (ev < n))
            mi = m.astype(jnp.int32)
            cs = plsc.cumsum(mi)
            pos = jnp.maximum(off + cs - 1, 0)
            plsc.store_scatter(dstlist, [pos], pk, mask=m)
            return off + cs[LANES - 1], ev + LANES

        nb, _ = lax.fori_loop(0, nit, body, (jnp.int32(0), iota))
        return nb

    def _patch(buf, lst, nb):
        def body(e, esplat):
            pks = plsc.load_gather(lst, [esplat])[0]
            rr = pks & (PB - 1)
            cc = lax.shift_right_logical(pks, 18)
            for d in range(DIM // LANES):
                buf[rr, pl.ds(d * LANES, LANES)] = (
                    ctable_v[cc, pl.ds(d * LANES, LANES)])
            return esplat + 1
        lax.fori_loop(0, nb, body, zeros)

    def _unpatch(buf, lst, nb):
        def body(e, esplat):
            pks = plsc.load_gather(lst, [esplat])[0]
            rr = pks & (PB - 1)
            for d in range(DIM // LANES):
                buf[rr, pl.ds(d * LANES, LANES)] = pvs[d]
            return esplat + 1
        lax.fori_loop(0, nb, body, zeros)

    def _fire(buf, blk_scalar, sem):
        pltpu.async_copy(
            buf, out_hbm.at[pl.ds(row0 + blk_scalar * PB, PB)], sem)

    def _drain(buf, sem):
        pltpu.make_async_copy(buf, out_hbm.at[pl.ds(row0, PB)], sem).wait()

    na0 = _blk_compact(zeros, blk_a)
    _patch(buf_a, blk_a, na0)
    _fire(buf_a, jnp.int32(0), sem_a)
    nb0 = _blk_compact(zeros + 1, blk_b)
    _patch(buf_b, blk_b, nb0)
    _fire(buf_b, jnp.int32(1), sem_b)

    def _step(t, carry):
        na_p, nb_p, bva, bvb = carry
        bva = bva + 2
        bvb = bvb + 2
        _drain(buf_a, sem_a)
        _unpatch(buf_a, blk_a, na_p)
        na = _blk_compact(bva, blk_a)
        _patch(buf_a, blk_a, na)
        _fire(buf_a, bva[0], sem_a)
        _drain(buf_b, sem_b)
        _unpatch(buf_b, blk_b, nb_p)
        nbn = _blk_compact(bvb, blk_b)
        _patch(buf_b, blk_b, nbn)
        _fire(buf_b, bvb[0], sem_b)
        return na, nbn, bva, bvb

    lax.fori_loop(1, NBLK // 2, _step, (na0, nb0, zeros, zeros + 1))

    _drain(buf_a, sem_a)
    _drain(buf_b, sem_b)


def kernel(edge_index, edge_feat, num_nodes, padding_emb, table0, table1, table2):
    nn = jnp.asarray(num_nodes, jnp.int32).reshape(1)
    ctable, pk = _prep(nn, edge_index.astype(jnp.int32),
                       edge_feat.astype(jnp.int32), table0, table1, table2)
    out = _sc_fill_scatter(pk, ctable, padding_emb)
    return out.reshape(N_NODES, N_NODES, DIM)
